# baseline (speedup 1.0000x reference)
import functools

import jax
import jax.numpy as jnp
from jax import lax
from jax.experimental import pallas as pl
from jax.experimental.pallas import tpu as pltpu
from jax.experimental.pallas import tpu_sc as plsc

NUM_WORKERS = 32
SAMPLES_PER_CHUNK = 8
NBUF = 2


def _gather_body(idx_hbm, table_hbm, out_hbm, idx_v, rows_v, sem_idx,
                 sem_gather, sem_out, *, chunks_per_worker, hist):
  wid = lax.axis_index("s") * 2 + lax.axis_index("c")
  samp0 = wid * (chunks_per_worker * SAMPLES_PER_CHUNK)
  n = chunks_per_worker

  def chunk_start(g):
    return pl.multiple_of(samp0 + g * SAMPLES_PER_CHUNK, SAMPLES_PER_CHUNK)

  def start_idx_load(g, b):
    pltpu.async_copy(idx_hbm.at[pl.ds(chunk_start(g), SAMPLES_PER_CHUNK)],
                     idx_v.at[b], sem_idx.at[b])

  def wait_idx_load(g, b):
    pltpu.make_async_copy(
        idx_hbm.at[pl.ds(chunk_start(g), SAMPLES_PER_CHUNK)], idx_v.at[b],
        sem_idx.at[b]).wait()

  def gather_descs(b):
    return [
        pltpu.make_async_copy(
            table_hbm.at[idx_v.at[b, i]],
            rows_v.at[b, i],
            sem_gather.at[b],
        ) for i in range(SAMPLES_PER_CHUNK)
    ]

  def out_desc(g, b):
    return pltpu.make_async_copy(
        rows_v.at[b],
        out_hbm.at[pl.ds(chunk_start(g), SAMPLES_PER_CHUNK)], sem_out.at[b])

  start_idx_load(0, 0)

  def body(g, carry):
    b = lax.rem(g, NBUF)
    @pl.when(g >= NBUF)
    def _():
      out_desc(g - NBUF, b).wait()

    wait_idx_load(g, b)
    for d in gather_descs(b):
      d.start()

    @pl.when(g + 1 < n)
    def _():
      start_idx_load(g + 1, 1 - b)

    for d in gather_descs(b):
      d.wait()
    out_desc(g, b).start()
    return carry

  lax.fori_loop(0, n, body, 0, unroll=False)

  for k in range(NBUF):
    g = n - NBUF + k
    out_desc(g, lax.rem(g, NBUF)).wait()


def kernel(indices, table):
  batch, hist = indices.shape
  _, embed_dim = table.shape
  assert batch % (NUM_WORKERS * SAMPLES_PER_CHUNK) == 0
  chunks_per_worker = batch // (NUM_WORKERS * SAMPLES_PER_CHUNK)

  mesh = plsc.VectorSubcoreMesh(core_axis_name="c", subcore_axis_name="s")
  gather = functools.partial(
      pl.kernel,
      mesh=mesh,
      out_type=jax.ShapeDtypeStruct((batch, hist, embed_dim), jnp.float32),
      scratch_types=[
          pltpu.VMEM((NBUF, SAMPLES_PER_CHUNK, hist), jnp.int32),
          pltpu.VMEM((NBUF, SAMPLES_PER_CHUNK, hist, embed_dim), jnp.float32),
          pltpu.SemaphoreType.DMA((NBUF,)),
          pltpu.SemaphoreType.DMA((NBUF,)),
          pltpu.SemaphoreType.DMA((NBUF,)),
      ],
      compiler_params=pltpu.CompilerParams(use_tc_tiling_on_sc=False),
  )(functools.partial(
      _gather_body, chunks_per_worker=chunks_per_worker, hist=hist))

  return gather(indices.astype(jnp.int32), table)
